# Initial kernel scaffold; baseline (speedup 1.0000x reference)
import jax
import jax.numpy as jnp
from jax.experimental import pallas as pl


def kernel(x, Wq, Wk, Wv, Wo, attn_norm_w, gate_w, gate_b, log_temp, r_norm_w, conv_w, Wg, Wu, Wd):
    raise NotImplementedError("write your pallas kernel here")


# speedup vs baseline: 1.5365x; 1.5365x over previous
import functools

import jax
import jax.numpy as jnp
from jax.experimental import pallas as pl
from jax.experimental.pallas import tpu as pltpu

EPS = 1.1920929e-07
NEG = -1e30
DH = 64


def _rms(x, w):
    return x * jax.lax.rsqrt(jnp.mean(x * x, axis=-1, keepdims=True) + EPS) * w


def _dot(a, b, dims):
    return jax.lax.dot_general(a, b, (dims, ((), ())),
                               preferred_element_type=jnp.float32)


def _cumsum_lanes(x):
    S = x.shape[1]
    lane = jax.lax.broadcasted_iota(jnp.int32, x.shape, 1)
    sh = 1
    while sh < S:
        r = pltpu.roll(x, sh, axis=1)
        x = x + jnp.where(lane >= sh, r, 0.0)
        sh *= 2
    return x


def _router_kernel(K, KP, x_ref, gw_ref, gb_ref, p_ref, aux_ref):
    S = x_ref.shape[0]
    l = _dot(gw_ref[...], x_ref[...], ((1,), (1,))) + gb_ref[...]
    p = jax.nn.sigmoid(l)
    pm = jnp.sum(p, axis=1, keepdims=True) / S
    sp = (pm - 0.1) ** 2
    ent = -(p * jnp.log(p + 1e-10) + (1.0 - p) * jnp.log(1.0 - p + 1e-10))
    aux_ref[...] = 0.1 * sp + 0.01 * (jnp.sum(ent, axis=1, keepdims=True) / S)
    u = jax.lax.bitcast_convert_type(l, jnp.int32)
    key = jnp.where(u >= 0, u, u ^ jnp.int32(0x7FFFFFFF))
    MIN32 = jnp.int32(-(2 ** 31))
    tb = jnp.zeros((1, 1), jnp.int32)
    for b in range(31, -1, -1):
        bit = MIN32 if b == 31 else jnp.int32(1 << b)
        cand = tb | bit
        thr = cand ^ MIN32
        cnt = jnp.sum(jnp.where(key >= thr, 1.0, 0.0), axis=1, keepdims=True)
        tb = jnp.where(cnt >= K, cand, tb)
    vk = tb ^ MIN32
    gt = key > vk
    eq = key == vk
    C = jnp.sum(jnp.where(gt, 1.0, 0.0), axis=1, keepdims=True)
    eqf = jnp.where(eq, 1.0, 0.0)
    eqpos = _cumsum_lanes(eqf) - eqf
    sel = jnp.where(gt, 1.0, jnp.where(eq & (eqpos < (K - C)), 1.0, 0.0))
    pos = _cumsum_lanes(sel) - sel
    rows = jax.lax.broadcasted_iota(jnp.int32, (KP, 1), 0).astype(jnp.float32)
    p_ref[...] = jnp.where((sel > 0.5) & (pos == rows), 1.0, 0.0)


def _kv_kernel(KVH, x_ref, anw_ref, wk_ref, wkr_ref, wv_ref, cos_ref, sin_ref,
               xn_ref, k_ref, v_ref):
    xn = _rms(x_ref[...], anw_ref[...])
    xn_ref[...] = xn
    k0 = _dot(xn, wk_ref[...], ((1,), (1,)))
    kr = _dot(xn, wkr_ref[...], ((1,), (1,)))
    cos = jnp.concatenate([cos_ref[...]] * KVH, axis=1)
    sin = jnp.concatenate([sin_ref[...]] * KVH, axis=1)
    k_ref[...] = k0 * cos + kr * sin
    v_ref[...] = _dot(xn, wv_ref[...], ((1,), (1,)))


def _qsel_kernel(H, p_ref, xn_ref, wq_ref, wqr_ref, cos_ref, sin_ref,
                 q_ref, idx_ref):
    P = p_ref[...]
    xs = _dot(P, xn_ref[...], ((1,), (0,)))
    q0 = _dot(xs, wq_ref[...], ((1,), (1,)))
    qr = _dot(xs, wqr_ref[...], ((1,), (1,)))
    cs = _dot(P, cos_ref[...], ((1,), (0,)))
    sn = _dot(P, sin_ref[...], ((1,), (0,)))
    cos = jnp.concatenate([cs] * H, axis=1)
    sin = jnp.concatenate([sn] * H, axis=1)
    q_ref[...] = q0 * cos + qr * sin
    lane = jax.lax.broadcasted_iota(jnp.int32, P.shape, 1).astype(jnp.float32)
    idx_ref[...] = jnp.sum(P * lane, axis=1, keepdims=True)


def _attn_kernel(scale, q_ref, k_ref, v_ref, idx_ref, o_ref):
    q = q_ref[0]
    k = k_ref[0]
    v = v_ref[0]
    s = _dot(q, k, ((1,), (1,))) * scale
    j = jax.lax.broadcasted_iota(jnp.int32, s.shape, 1).astype(jnp.float32)
    s = jnp.where(idx_ref[...] >= j, s, NEG)
    m = jnp.max(s, axis=1, keepdims=True)
    e = jnp.exp(s - m)
    den = jnp.sum(e, axis=1, keepdims=True)
    o = _dot(e, v, ((1,), (0,)))
    o_ref[0] = o / den


def _oproj_kernel(o_ref, wo_ref, out_ref):
    out_ref[...] = _dot(o_ref[...], wo_ref[...], ((1,), (1,)))


def _scatter_kernel(p_ref, op_ref, x_ref, rnw_ref, h_ref, hn_ref):
    h = x_ref[...] + _dot(p_ref[...], op_ref[...], ((0,), (0,)))
    h_ref[...] = h
    hn_ref[...] = _rms(h, rnw_ref[...])


def _conv_kernel(NSB, hp_ref, h_ref, hx_ref, w0_ref, w1_ref, w2_ref, hc_ref):
    s = pl.program_id(0)
    hn = h_ref[...]
    prev_last = jnp.where(s > 0, hp_ref[-1:, :], 0.0)
    next_first = jnp.where(s < NSB - 1, hx_ref[:1, :], 0.0)
    prev = jnp.concatenate([prev_last, hn[:-1, :]], axis=0)
    nxt = jnp.concatenate([hn[1:, :], next_first], axis=0)
    hc_ref[...] = w0_ref[...] * prev + w1_ref[...] * hn + w2_ref[...] * nxt


def _mlp_kernel(hc_ref, h_ref, wg_ref, wu_ref, wd_ref, y_ref):
    hc = hc_ref[...]
    g = _dot(hc, wg_ref[...], ((1,), (1,)))
    u = _dot(hc, wu_ref[...], ((1,), (1,)))
    a = g * jax.nn.sigmoid(g) * u
    y_ref[...] = h_ref[...] + _dot(a, wd_ref[...], ((1,), (1,)))


def _rope_rows(W, nheads):
    Wh = W.reshape(nheads, DH, -1)
    return jnp.concatenate([-Wh[:, DH // 2:], Wh[:, :DH // 2]], axis=1).reshape(W.shape)


def kernel(x, Wq, Wk, Wv, Wo, attn_norm_w, gate_w, gate_b, log_temp,
           r_norm_w, conv_w, Wg, Wu, Wd):
    B, S, D = x.shape
    H = Wq.shape[0] // DH
    KVH = Wk.shape[0] // DH
    HID = Wg.shape[0]
    KQ = max(1, int(S * 0.1))
    KP = ((KQ + 127) // 128) * 128
    SB = min(512, S)
    NSB = S // SB
    GRP = H // KVH

    x2 = x.reshape(S, D)
    f32 = jnp.float32

    inv_freq = 1.0 / (10000.0 ** (jnp.arange(0, DH, 2, dtype=f32) / DH))
    fr = jnp.outer(jnp.arange(S, dtype=f32), inv_freq)
    emb = jnp.concatenate([fr, fr], axis=-1)
    cos64 = jnp.cos(emb)
    sin64 = jnp.sin(emb)
    WqR = _rope_rows(Wq, H)
    WkR = _rope_rows(Wk, KVH)
    anw = attn_norm_w.reshape(1, D)
    rnw = r_norm_w.reshape(1, D)
    gw = gate_w.reshape(1, D)
    gb = gate_b.reshape(1, 1)
    w0 = conv_w[:, 0, 0].reshape(1, D)
    w1 = conv_w[:, 0, 1].reshape(1, D)
    w2 = conv_w[:, 0, 2].reshape(1, D)

    full = lambda shp: pl.BlockSpec(shp, lambda *_: tuple(0 for _ in shp))

    P, aux = pl.pallas_call(
        functools.partial(_router_kernel, KQ, KP),
        out_shape=[jax.ShapeDtypeStruct((KP, S), f32),
                   jax.ShapeDtypeStruct((1, 1), f32)],
        in_specs=[full((S, D)), full((1, D)), full((1, 1))],
        out_specs=[full((KP, S)), full((1, 1))],
    )(x2, gw, gb)

    xn, k_rot, v = pl.pallas_call(
        functools.partial(_kv_kernel, KVH),
        grid=(NSB,),
        out_shape=[jax.ShapeDtypeStruct((S, D), f32),
                   jax.ShapeDtypeStruct((S, KVH * DH), f32),
                   jax.ShapeDtypeStruct((S, KVH * DH), f32)],
        in_specs=[pl.BlockSpec((SB, D), lambda s: (s, 0)),
                  full((1, D)),
                  full((KVH * DH, D)), full((KVH * DH, D)), full((KVH * DH, D)),
                  pl.BlockSpec((SB, DH), lambda s: (s, 0)),
                  pl.BlockSpec((SB, DH), lambda s: (s, 0))],
        out_specs=[pl.BlockSpec((SB, D), lambda s: (s, 0)),
                   pl.BlockSpec((SB, KVH * DH), lambda s: (s, 0)),
                   pl.BlockSpec((SB, KVH * DH), lambda s: (s, 0))],
    )(x2, anw, Wk, WkR, Wv, cos64, sin64)

    q_sel, idx_f = pl.pallas_call(
        functools.partial(_qsel_kernel, H),
        out_shape=[jax.ShapeDtypeStruct((KP, D), f32),
                   jax.ShapeDtypeStruct((KP, 1), f32)],
        in_specs=[full((KP, S)), full((S, D)), full((D, D)), full((D, D)),
                  full((S, DH)), full((S, DH))],
        out_specs=[full((KP, D)), full((KP, 1))],
    )(P, xn, Wq, WqR, cos64, sin64)

    q3 = q_sel.reshape(KP, H, DH).transpose(1, 0, 2)
    k3 = k_rot.reshape(S, KVH, DH).transpose(1, 0, 2)
    v3 = v.reshape(S, KVH, DH).transpose(1, 0, 2)
    o3 = pl.pallas_call(
        functools.partial(_attn_kernel, 1.0 / (DH ** 0.5)),
        grid=(H,),
        out_shape=jax.ShapeDtypeStruct((H, KP, DH), f32),
        in_specs=[pl.BlockSpec((1, KP, DH), lambda h: (h, 0, 0)),
                  pl.BlockSpec((1, S, DH), lambda h: (h // GRP, 0, 0)),
                  pl.BlockSpec((1, S, DH), lambda h: (h // GRP, 0, 0)),
                  full((KP, 1))],
        out_specs=pl.BlockSpec((1, KP, DH), lambda h: (h, 0, 0)),
    )(q3, k3, v3, idx_f)
    o_sel = o3.transpose(1, 0, 2).reshape(KP, H * DH)

    o_proj = pl.pallas_call(
        _oproj_kernel,
        out_shape=jax.ShapeDtypeStruct((KP, D), f32),
        in_specs=[full((KP, D)), full((D, D))],
        out_specs=full((KP, D)),
    )(o_sel, Wo)

    h, hn = pl.pallas_call(
        _scatter_kernel,
        grid=(NSB,),
        out_shape=[jax.ShapeDtypeStruct((S, D), f32),
                   jax.ShapeDtypeStruct((S, D), f32)],
        in_specs=[pl.BlockSpec((KP, SB), lambda s: (0, s)),
                  full((KP, D)),
                  pl.BlockSpec((SB, D), lambda s: (s, 0)),
                  full((1, D))],
        out_specs=[pl.BlockSpec((SB, D), lambda s: (s, 0)),
                   pl.BlockSpec((SB, D), lambda s: (s, 0))],
    )(P, o_proj, x2, rnw)

    hc = pl.pallas_call(
        functools.partial(_conv_kernel, NSB),
        grid=(NSB,),
        out_shape=jax.ShapeDtypeStruct((S, D), f32),
        in_specs=[pl.BlockSpec((SB, D), lambda s: (jnp.maximum(s - 1, 0), 0)),
                  pl.BlockSpec((SB, D), lambda s: (s, 0)),
                  pl.BlockSpec((SB, D), lambda s: (jnp.minimum(s + 1, NSB - 1), 0)),
                  full((1, D)), full((1, D)), full((1, D))],
        out_specs=pl.BlockSpec((SB, D), lambda s: (s, 0)),
    )(hn, hn, hn, w0, w1, w2)

    SBM = min(256, S)
    y = pl.pallas_call(
        _mlp_kernel,
        grid=(S // SBM,),
        out_shape=jax.ShapeDtypeStruct((S, D), f32),
        in_specs=[pl.BlockSpec((SBM, D), lambda s: (s, 0)),
                  pl.BlockSpec((SBM, D), lambda s: (s, 0)),
                  full((HID, D)), full((HID, D)), full((D, HID))],
        out_specs=pl.BlockSpec((SBM, D), lambda s: (s, 0)),
    )(hc, h, Wg, Wu, Wd)

    return y.reshape(B, S, D), aux[0, 0]
